# baseline (speedup 1.0000x reference)
import jax
import jax.numpy as jnp
from jax import lax
from jax.experimental import pallas as pl
from jax.experimental.pallas import tpu as pltpu
from jax.experimental.pallas import tpu_sc as plsc

N = 10000
E = 320000
D = 128

NC = 2
NS = 16
NW = NC * NS

NB = 400
N_BLOCKS = N // NB


def _bf16_bits(x):
    b = lax.bitcast_convert_type(x, jnp.int32)
    b = b + jnp.int32(0x7FFF) + (lax.shift_right_logical(b, 16) & jnp.int32(1))
    return b & jnp.int32(-65536)


def _pack2(hi_f32, lo_f32):
    return _bf16_bits(hi_f32) | lax.shift_right_logical(_bf16_bits(lo_f32), 16)


def _unpack_hi(i32):
    return lax.bitcast_convert_type(i32 & jnp.int32(-65536), jnp.float32)


def _unpack_lo(i32):
    return lax.bitcast_convert_type(lax.shift_left(i32, 16), jnp.float32)


PB = 80
P_BLOCKS = N // PB
NR8 = N * 8


def _proj_body(h_ref, wall_ref, rel8_ref, wrel_ref, brel_ref, w2_ref, b2_ref,
               t_ref, pgd_ref, reltab_ref):
    h = h_ref[...]
    p = jnp.dot(h, wall_ref[...], preferred_element_type=jnp.float32)
    pm, pgs, pgd = p[:, :D], p[:, D:2 * D], p[:, 2 * D:]
    pgd_ref[...] = pgd
    rt = (jnp.dot(rel8_ref[...], wrel_ref[...], preferred_element_type=jnp.float32)
          + brel_ref[...])
    relm = rt[:, :D]
    u = jax.nn.gelu(pm[:, None, :] + relm[None, :, :]).reshape(PB * 8, D)
    m = (jnp.dot(u.astype(jnp.bfloat16), w2_ref[...],
                 preferred_element_type=jnp.float32) + b2_ref[...])
    pgs_b = jnp.broadcast_to(pgs[:, None, :], (PB, 8, D)).reshape(PB * 8, D)
    t_ref[...] = _pack2(m, pgs_b)

    @pl.when(pl.program_id(0) == 0)
    def _():
        reltab_ref[...] = rt


def _proj_call(h, wall, rel8, wrel, brel, w2, b2):
    return pl.pallas_call(
        _proj_body,
        grid=(P_BLOCKS,),
        in_specs=[
            pl.BlockSpec((PB, D), lambda i: (i, 0)),
            pl.BlockSpec((D, 3 * D), lambda i: (0, 0)),
            pl.BlockSpec((8, D), lambda i: (0, 0)),
            pl.BlockSpec((D, 2 * D), lambda i: (0, 0)),
            pl.BlockSpec((1, 2 * D), lambda i: (0, 0)),
            pl.BlockSpec((D, D), lambda i: (0, 0)),
            pl.BlockSpec((1, D), lambda i: (0, 0)),
        ],
        out_specs=[
            pl.BlockSpec((PB * 8, D), lambda i: (i, 0)),
            pl.BlockSpec((PB, D), lambda i: (i, 0)),
            pl.BlockSpec((8, 2 * D), lambda i: (0, 0)),
        ],
        out_shape=[
            jax.ShapeDtypeStruct((NR8, D), jnp.int32),
            jax.ShapeDtypeStruct((N, D), jnp.float32),
            jax.ShapeDtypeStruct((8, 2 * D), jnp.float32),
        ],
    )(h, wall, rel8, wrel, brel, w2, b2)


def _eidx_body(src_ref, rel_ref, out_ref):
    out_ref[...] = src_ref[...] * 8 + rel_ref[...]


def _eidx_call(src3, rel3flat):
    return pl.pallas_call(
        _eidx_body,
        out_shape=jax.ShapeDtypeStruct((1, 1, E), jnp.int32),
    )(src3, rel3flat)


GC = 80
E0 = 158720


_MASKHI = -65536


def _gather_call(t, pgd, src3, dst3, epw):
    niters = epw // GC
    eh = epw * NW
    mesh = plsc.VectorSubcoreMesh(core_axis_name="c", subcore_axis_name="s")

    def body(t_hbm, pgd_hbm, src3_hbm, dst3_hbm, out_hbm,
             sidx, didx, sbufs, dbufs, semt, semp):
        wid = lax.axis_index("s") * NC + lax.axis_index("c")
        pltpu.sync_copy(src3_hbm.at[wid], sidx)
        pltpu.sync_copy(dst3_hbm.at[wid], didx)
        base = wid * epw

        def fire(it, p):
            pltpu.async_copy(t_hbm.at[sidx.at[it]], sbufs.at[p], semt)
            pltpu.async_copy(pgd_hbm.at[didx.at[it]], dbufs.at[p], semp)

        fire(0, 0)

        def step(it, carry):
            p = lax.rem(it, 2)

            @pl.when(it + 1 < niters)
            def _():
                fire(it + 1, 1 - p)

            pltpu.make_async_copy(t_hbm.at[sidx.at[it]], sbufs.at[p], semt).wait()
            pltpu.make_async_copy(pgd_hbm.at[didx.at[it]], dbufs.at[p], semp).wait()

            def comb(e, c2):
                for cc in range(D // 16):
                    sl = pl.ds(cc * 16, 16)
                    ti = sbufs[p, e, sl]
                    s = (lax.bitcast_convert_type(lax.shift_left(ti, 16),
                                                  jnp.float32)
                         + dbufs[p, e, sl])
                    sbufs[p, e, sl] = (
                        (ti & _MASKHI)
                        | lax.shift_right_logical(_bf16_bits(s), 16))
                return c2

            lax.fori_loop(0, GC, comb, 0)
            off = pl.multiple_of(base + it * GC, 8)
            pltpu.sync_copy(sbufs.at[p], out_hbm.at[pl.ds(off, GC)])
            return carry

        lax.fori_loop(0, niters, step, 0)

    f = pl.kernel(
        body,
        out_type=jax.ShapeDtypeStruct((eh, D), jnp.int32),
        mesh=mesh,
        scratch_types=[
            pltpu.VMEM((niters, GC), jnp.int32),
            pltpu.VMEM((niters, GC), jnp.int32),
            pltpu.VMEM((2, GC, D), jnp.int32),
            pltpu.VMEM((2, GC, D), jnp.float32),
            pltpu.SemaphoreType.DMA,
            pltpu.SemaphoreType.DMA,
        ],
    )
    return f(t, pgd, src3, dst3)


EB = 512
E_BLOCKS = E // EB


def _edge_body(gs_ref, rel_ref, reltab_ref, gw2_ref, gb2_ref, g_ref):
    ids = rel_ref[0, 0, :]
    onehot = (ids[:, None] == lax.broadcasted_iota(jnp.int32, (EB, 8), 1)
              ).astype(jnp.float32)
    relg = reltab_ref[...][:, D:]
    addend = jnp.dot(onehot, relg, preferred_element_type=jnp.float32)
    v = jax.nn.gelu(_unpack_lo(gs_ref[...]) + addend)
    gsc = jnp.sum(v * gw2_ref[...], axis=-1, keepdims=True) + gb2_ref[...]
    g_ref[...] = jax.nn.sigmoid(gsc)


def _edge_call(gs, rel3, reltab, gw2row, gb2):
    eh = gs.shape[0]
    return pl.pallas_call(
        _edge_body,
        grid=(eh // EB,),
        in_specs=[
            pl.BlockSpec((EB, D), lambda i: (i, 0)),
            pl.BlockSpec((1, 1, EB), lambda i: (i, 0, 0)),
            pl.BlockSpec((8, 2 * D), lambda i: (0, 0)),
            pl.BlockSpec((1, D), lambda i: (0, 0)),
            pl.BlockSpec((1, 1), lambda i: (0, 0)),
        ],
        out_specs=pl.BlockSpec((EB, 1), lambda i: (i, 0)),
        out_shape=jax.ShapeDtypeStruct((eh, 1), jnp.float32),
    )(gs, rel3, reltab, gw2row, gb2)


NP = 10240
RPT = NP // NS


def _scatter_call(gs, g3, dst3, ept):
    niters = ept // GC
    mesh = plsc.VectorSubcoreMesh(core_axis_name="c", subcore_axis_name="s")

    def body(gs_hbm, g3_hbm, dst3_hbm, parts_hbm,
             idx, gbuf, rbufs, abufs, agg_sh, semr):
        c = lax.axis_index("c")
        s = lax.axis_index("s")
        wid = c * NS + s

        def zb(k, c2):
            abufs[0, k // 8, pl.ds((k % 8) * 16, 16)] = jnp.zeros(
                (16,), jnp.float32)
            return c2

        lax.fori_loop(0, GC * 8, zb, 0)
        for j in range(RPT // GC):
            pltpu.sync_copy(abufs.at[0],
                            agg_sh.at[pl.ds(s * RPT + j * GC, GC)])
        pltpu.sync_copy(dst3_hbm.at[wid], idx)
        pltpu.sync_copy(g3_hbm.at[wid], gbuf)
        base = wid * ept
        plsc.subcore_barrier()

        def fire(it, p):
            off = pl.multiple_of(base + it * GC, 8)
            pltpu.async_copy(gs_hbm.at[pl.ds(off, GC)], rbufs.at[p], semr)

        fire(0, 0)

        def step(it, carry):
            p = lax.rem(it, 2)

            @pl.when(it + 1 < niters)
            def _():
                fire(it + 1, 1 - p)

            pltpu.make_async_copy(gs_hbm.at[pl.ds(base, GC)],
                                  rbufs.at[p], semr).wait()

            def ge_grp(eg, c2):
                g16 = gbuf[it, pl.ds(eg * 16, 16)]
                for j in range(16):
                    gv = g16[j]
                    e = eg * 16 + j
                    for cc in range(D // 16):
                        sl = pl.ds(cc * 16, 16)
                        abufs[0, e, sl] = gv * lax.bitcast_convert_type(
                            rbufs[p, e, sl] & _MASKHI, jnp.float32)
                return c2

            lax.fori_loop(0, GC // 16, ge_grp, 0)
            pltpu.sync_copy(abufs.at[0], agg_sh.at[idx.at[it]], add=True)
            return carry

        lax.fori_loop(0, niters, step, 0)
        plsc.subcore_barrier()
        pltpu.sync_copy(agg_sh.at[pl.ds(s * RPT, RPT)],
                        parts_hbm.at[c, pl.ds(s * RPT, RPT)])

    f = pl.kernel(
        body,
        out_type=jax.ShapeDtypeStruct((NC, NP, D), jnp.float32),
        mesh=mesh,
        scratch_types=[
            pltpu.VMEM((niters, GC), jnp.int32),
            pltpu.VMEM((niters, GC), jnp.float32),
            pltpu.VMEM((2, GC, D), jnp.int32),
            pltpu.VMEM((1, GC, D), jnp.float32),
            pltpu.VMEM_SHARED((NP, D), jnp.float32),
            pltpu.SemaphoreType.DMA,
        ],
    )
    return f(gs, g3, dst3)


def _ln_body(h_ref, p00_ref, p01_ref, p10_ref, p11_ref, g_ref, b_ref, out_ref):
    x = (h_ref[...] + p00_ref[0] + p01_ref[0] + p10_ref[0] + p11_ref[0])
    mu = jnp.mean(x, axis=-1, keepdims=True)
    xc = x - mu
    var = jnp.mean(xc * xc, axis=-1, keepdims=True)
    out_ref[...] = xc * lax.rsqrt(var + 1e-5) * g_ref[...] + b_ref[...]


def _ln_call(h, parts0, parts1, g, b):
    return pl.pallas_call(
        _ln_body,
        grid=(N_BLOCKS,),
        in_specs=[
            pl.BlockSpec((NB, D), lambda i: (i, 0)),
            pl.BlockSpec((1, NB, D), lambda i: (0, i, 0)),
            pl.BlockSpec((1, NB, D), lambda i: (1, i, 0)),
            pl.BlockSpec((1, NB, D), lambda i: (0, i, 0)),
            pl.BlockSpec((1, NB, D), lambda i: (1, i, 0)),
            pl.BlockSpec((1, D), lambda i: (0, 0)),
            pl.BlockSpec((1, D), lambda i: (0, 0)),
        ],
        out_specs=pl.BlockSpec((NB, D), lambda i: (i, 0)),
        out_shape=jax.ShapeDtypeStruct((N, D), jnp.float32),
    )(h, parts0, parts0, parts1, parts1, g, b)


def _pool_body(h_ref, pw1_ref, pb1_ref, pw2_ref, pb2_ref, out_ref,
               sum_ref, max_ref):
    i = pl.program_id(0)

    @pl.when(i == 0)
    def _():
        sum_ref[...] = jnp.zeros_like(sum_ref)
        max_ref[...] = jnp.full_like(max_ref, -jnp.inf)

    blk = h_ref[...]
    sum_ref[...] += jnp.broadcast_to(jnp.sum(blk, axis=0, keepdims=True), (8, D))
    max_ref[...] = jnp.maximum(
        max_ref[...], jnp.broadcast_to(jnp.max(blk, axis=0, keepdims=True), (8, D)))

    @pl.when(i == N_BLOCKS - 1)
    def _():
        mean8 = sum_ref[...] * (1.0 / N)
        pin = jnp.concatenate([mean8, max_ref[...]], axis=-1)
        hdn = jax.nn.gelu(
            jnp.dot(pin, pw1_ref[...], preferred_element_type=jnp.float32)
            + pb1_ref[...])
        out_ref[...] = (
            jnp.dot(hdn, pw2_ref[...], preferred_element_type=jnp.float32)
            + pb2_ref[...])


def _pool_call(h, pw1, pb1, pw2, pb2):
    return pl.pallas_call(
        _pool_body,
        grid=(N_BLOCKS,),
        in_specs=[
            pl.BlockSpec((NB, D), lambda i: (i, 0)),
            pl.BlockSpec((2 * D, D), lambda i: (0, 0)),
            pl.BlockSpec((1, D), lambda i: (0, 0)),
            pl.BlockSpec((D, D), lambda i: (0, 0)),
            pl.BlockSpec((1, D), lambda i: (0, 0)),
        ],
        out_specs=pl.BlockSpec((8, D), lambda i: (0, 0)),
        out_shape=jax.ShapeDtypeStruct((8, D), jnp.float32),
        scratch_shapes=[
            pltpu.VMEM((8, D), jnp.float32),
            pltpu.VMEM((8, D), jnp.float32),
        ],
    )(h, pw1, pb1, pw2, pb2)


def kernel(node_states, edge_index, rel_ids, rel_emb,
           msg_W1, msg_b1, msg_W2, msg_b2,
           gate_W1, gate_b1, gate_W2, gate_b2,
           ln_g, ln_b, pool_W1, pool_b1, pool_W2, pool_b2):
    src = edge_index[0]
    dst = edge_index[1]
    rel8 = jnp.pad(rel_emb, ((0, 8 - rel_emb.shape[0]), (0, 0)))
    eidx = _eidx_call(src.reshape(1, 1, E), rel_ids.reshape(1, 1, E)).reshape(E)

    bounds = [(0, E0), (E0, E)]
    chunks = []
    for lo, hi in bounds:
        eh = hi - lo
        epw = eh // NW
        chunks.append(dict(
            epw=epw,
            e3=eidx[lo:hi].reshape(NW, epw // GC, GC),
            d3=dst[lo:hi].reshape(NW, epw // GC, GC),
            rel3=rel_ids[lo:hi].reshape(eh // EB, 1, EB),
        ))

    h = node_states
    L = msg_W1.shape[0]
    for l in range(L):
        wall = jnp.concatenate(
            [msg_W1[l][:D], gate_W1[l][D:2 * D], gate_W1[l][:D]], axis=1)
        wrel = jnp.concatenate([msg_W1[l][D:], gate_W1[l][2 * D:]], axis=1)
        brel = jnp.concatenate([msg_b1[l], gate_b1[l]])[None, :]
        t, pgd, reltab = _proj_call(h, wall, rel8, wrel, brel,
                                    msg_W2[l].astype(jnp.bfloat16),
                                    msg_b2[l][None, :])
        parts = []
        for ck in chunks:
            gs = _gather_call(t, pgd, ck["e3"], ck["d3"], ck["epw"])
            g = _edge_call(gs, ck["rel3"], reltab,
                           gate_W2[l].T, gate_b2[l][None, :])
            g3 = g.reshape(NW, ck["epw"] // GC, GC)
            parts.append(_scatter_call(gs, g3, ck["d3"], ck["epw"]))
        h = _ln_call(h, parts[0], parts[1], ln_g[l][None, :], ln_b[l][None, :])

    pooled = _pool_call(h, pool_W1, pool_b1[None, :], pool_W2, pool_b2[None, :])
    return jnp.concatenate([h, pooled[:1]], axis=0)


# speedup vs baseline: 1.5647x; 1.5647x over previous
import jax
import jax.numpy as jnp
from jax import lax
from jax.experimental import pallas as pl
from jax.experimental.pallas import tpu as pltpu
from jax.experimental.pallas import tpu_sc as plsc

N = 10000
E = 320000
D = 128

NC = 2
NS = 16
NW = NC * NS

NB = 400
N_BLOCKS = N // NB


def _bf16_bits(x):
    b = lax.bitcast_convert_type(x, jnp.int32)
    b = b + jnp.int32(0x7FFF) + (lax.shift_right_logical(b, 16) & jnp.int32(1))
    return b & jnp.int32(-65536)


def _pack2(hi_f32, lo_f32):
    return _bf16_bits(hi_f32) | lax.shift_right_logical(_bf16_bits(lo_f32), 16)


def _unpack_hi(i32):
    return lax.bitcast_convert_type(i32 & jnp.int32(-65536), jnp.float32)


def _unpack_lo(i32):
    return lax.bitcast_convert_type(lax.shift_left(i32, 16), jnp.float32)


PB = 80
P_BLOCKS = N // PB
NR8 = N * 8


def _proj_body(h_ref, wall_ref, rel8_ref, wrel_ref, brel_ref, w2_ref, b2_ref,
               t_ref, pgd_ref, reltab_ref):
    h = h_ref[...]
    p = jnp.dot(h, wall_ref[...], preferred_element_type=jnp.float32)
    pm, pgs, pgd = p[:, :D], p[:, D:2 * D], p[:, 2 * D:]
    pgd_ref[...] = pgd
    rt = (jnp.dot(rel8_ref[...], wrel_ref[...], preferred_element_type=jnp.float32)
          + brel_ref[...])
    relm = rt[:, :D]
    u = jax.nn.gelu(pm[:, None, :] + relm[None, :, :]).reshape(PB * 8, D)
    m = (jnp.dot(u.astype(jnp.bfloat16), w2_ref[...],
                 preferred_element_type=jnp.float32) + b2_ref[...])
    pgs_b = jnp.broadcast_to(pgs[:, None, :], (PB, 8, D)).reshape(PB * 8, D)
    t_ref[...] = _pack2(m, pgs_b)

    @pl.when(pl.program_id(0) == 0)
    def _():
        reltab_ref[...] = rt


def _proj_call(h, wall, rel8, wrel, brel, w2, b2):
    return pl.pallas_call(
        _proj_body,
        grid=(P_BLOCKS,),
        in_specs=[
            pl.BlockSpec((PB, D), lambda i: (i, 0)),
            pl.BlockSpec((D, 3 * D), lambda i: (0, 0)),
            pl.BlockSpec((8, D), lambda i: (0, 0)),
            pl.BlockSpec((D, 2 * D), lambda i: (0, 0)),
            pl.BlockSpec((1, 2 * D), lambda i: (0, 0)),
            pl.BlockSpec((D, D), lambda i: (0, 0)),
            pl.BlockSpec((1, D), lambda i: (0, 0)),
        ],
        out_specs=[
            pl.BlockSpec((PB * 8, D), lambda i: (i, 0)),
            pl.BlockSpec((PB, D), lambda i: (i, 0)),
            pl.BlockSpec((8, 2 * D), lambda i: (0, 0)),
        ],
        out_shape=[
            jax.ShapeDtypeStruct((NR8, D), jnp.int32),
            jax.ShapeDtypeStruct((N, D), jnp.float32),
            jax.ShapeDtypeStruct((8, 2 * D), jnp.float32),
        ],
    )(h, wall, rel8, wrel, brel, w2, b2)


def _eidx_body(src_ref, rel_ref, out_ref):
    out_ref[...] = src_ref[...] * 8 + rel_ref[...]


def _eidx_call(src3, rel3flat):
    return pl.pallas_call(
        _eidx_body,
        out_shape=jax.ShapeDtypeStruct((1, 1, E), jnp.int32),
    )(src3, rel3flat)


GC = 80
E0 = 158720


_MASKHI = -65536


def _gather_call(t, pgd, src3, dst3, epw):
    niters = epw // GC
    eh = epw * NW
    mesh = plsc.VectorSubcoreMesh(core_axis_name="c", subcore_axis_name="s")

    def body(t_hbm, pgd_hbm, src3_hbm, dst3_hbm, gsrc_hbm, gdst_hbm,
             sidx, didx, sbufs, dbufs, semt, semp):
        wid = lax.axis_index("s") * NC + lax.axis_index("c")
        pltpu.sync_copy(src3_hbm.at[wid], sidx)
        pltpu.sync_copy(dst3_hbm.at[wid], didx)
        base = wid * epw

        def fire(it, p):
            pltpu.async_copy(t_hbm.at[sidx.at[it]], sbufs.at[p], semt)
            pltpu.async_copy(pgd_hbm.at[didx.at[it]], dbufs.at[p], semp)

        fire(0, 0)

        def step(it, carry):
            p = lax.rem(it, 2)

            @pl.when(it + 1 < niters)
            def _():
                fire(it + 1, 1 - p)

            pltpu.make_async_copy(t_hbm.at[sidx.at[it]], sbufs.at[p], semt).wait()
            pltpu.make_async_copy(pgd_hbm.at[didx.at[it]], dbufs.at[p], semp).wait()
            off = pl.multiple_of(base + it * GC, 8)
            pltpu.sync_copy(sbufs.at[p], gsrc_hbm.at[pl.ds(off, GC)])
            pltpu.sync_copy(dbufs.at[p], gdst_hbm.at[pl.ds(off, GC)])
            return carry

        lax.fori_loop(0, niters, step, 0)

    f = pl.kernel(
        body,
        out_type=[
            jax.ShapeDtypeStruct((eh, D), jnp.int32),
            jax.ShapeDtypeStruct((eh, D), jnp.float32),
        ],
        mesh=mesh,
        scratch_types=[
            pltpu.VMEM((niters, GC), jnp.int32),
            pltpu.VMEM((niters, GC), jnp.int32),
            pltpu.VMEM((2, GC, D), jnp.int32),
            pltpu.VMEM((2, GC, D), jnp.float32),
            pltpu.SemaphoreType.DMA,
            pltpu.SemaphoreType.DMA,
        ],
    )
    return f(t, pgd, src3, dst3)


EB = 512
E_BLOCKS = E // EB


def _edge_body(gsrc_ref, gdst_ref, rel_ref, reltab_ref,
               gw2_ref, gb2_ref, ge_ref):
    ids = rel_ref[0, 0, :]
    onehot = (ids[:, None] == lax.broadcasted_iota(jnp.int32, (EB, 8), 1)
              ).astype(jnp.float32)
    addend = jnp.dot(onehot, reltab_ref[...], preferred_element_type=jnp.float32)
    gi = gsrc_ref[...]
    m = _unpack_hi(gi)
    v = jax.nn.gelu(gdst_ref[...] + _unpack_lo(gi) + addend[:, D:])
    gsc = jnp.sum(v * gw2_ref[...], axis=-1, keepdims=True) + gb2_ref[...]
    ge_ref[...] = jax.nn.sigmoid(gsc) * m


def _edge_call(gsrc, gdst, rel3, reltab, gw2row, gb2):
    eh = gsrc.shape[0]
    return pl.pallas_call(
        _edge_body,
        grid=(eh // EB,),
        in_specs=[
            pl.BlockSpec((EB, D), lambda i: (i, 0)),
            pl.BlockSpec((EB, D), lambda i: (i, 0)),
            pl.BlockSpec((1, 1, EB), lambda i: (i, 0, 0)),
            pl.BlockSpec((8, 2 * D), lambda i: (0, 0)),
            pl.BlockSpec((1, D), lambda i: (0, 0)),
            pl.BlockSpec((1, 1), lambda i: (0, 0)),
        ],
        out_specs=pl.BlockSpec((EB, D), lambda i: (i, 0)),
        out_shape=jax.ShapeDtypeStruct((eh, D), jnp.float32),
    )(gsrc, gdst, rel3, reltab, gw2row, gb2)


NP = 10240
RPT = NP // NS


def _scatter_call(ge, dst3, zeros_nd, ept):
    niters = ept // GC
    mesh = plsc.VectorSubcoreMesh(core_axis_name="c", subcore_axis_name="s")

    def body(ge_hbm, dst3_hbm, zeros_hbm, parts_hbm, idx, rbufs, agg_sh, semr):
        c = lax.axis_index("c")
        s = lax.axis_index("s")
        wid = c * NS + s
        pltpu.sync_copy(zeros_hbm.at[pl.ds(s * RPT, RPT)],
                        agg_sh.at[pl.ds(s * RPT, RPT)])
        pltpu.sync_copy(dst3_hbm.at[wid], idx)
        base = wid * ept
        plsc.subcore_barrier()

        def fire(it, p):
            off = pl.multiple_of(base + it * GC, 8)
            pltpu.async_copy(ge_hbm.at[pl.ds(off, GC)], rbufs.at[p], semr)

        fire(0, 0)

        def step(it, carry):
            p = lax.rem(it, 2)

            @pl.when(it + 1 < niters)
            def _():
                fire(it + 1, 1 - p)

            pltpu.make_async_copy(ge_hbm.at[pl.ds(base, GC)],
                                  rbufs.at[p], semr).wait()
            pltpu.sync_copy(rbufs.at[p], agg_sh.at[idx.at[it]], add=True)
            return carry

        lax.fori_loop(0, niters, step, 0)
        plsc.subcore_barrier()
        pltpu.sync_copy(agg_sh.at[pl.ds(s * RPT, RPT)],
                        parts_hbm.at[c, pl.ds(s * RPT, RPT)])

    f = pl.kernel(
        body,
        out_type=jax.ShapeDtypeStruct((NC, NP, D), jnp.float32),
        mesh=mesh,
        scratch_types=[
            pltpu.VMEM((niters, GC), jnp.int32),
            pltpu.VMEM((2, GC, D), jnp.float32),
            pltpu.VMEM_SHARED((NP, D), jnp.float32),
            pltpu.SemaphoreType.DMA,
        ],
    )
    return f(ge, dst3, zeros_nd)


def _ln_body(h_ref, p00_ref, p01_ref, p10_ref, p11_ref, g_ref, b_ref, out_ref):
    x = (h_ref[...] + p00_ref[0] + p01_ref[0] + p10_ref[0] + p11_ref[0])
    mu = jnp.mean(x, axis=-1, keepdims=True)
    xc = x - mu
    var = jnp.mean(xc * xc, axis=-1, keepdims=True)
    out_ref[...] = xc * lax.rsqrt(var + 1e-5) * g_ref[...] + b_ref[...]


def _ln_call(h, parts0, parts1, g, b):
    return pl.pallas_call(
        _ln_body,
        grid=(N_BLOCKS,),
        in_specs=[
            pl.BlockSpec((NB, D), lambda i: (i, 0)),
            pl.BlockSpec((1, NB, D), lambda i: (0, i, 0)),
            pl.BlockSpec((1, NB, D), lambda i: (1, i, 0)),
            pl.BlockSpec((1, NB, D), lambda i: (0, i, 0)),
            pl.BlockSpec((1, NB, D), lambda i: (1, i, 0)),
            pl.BlockSpec((1, D), lambda i: (0, 0)),
            pl.BlockSpec((1, D), lambda i: (0, 0)),
        ],
        out_specs=pl.BlockSpec((NB, D), lambda i: (i, 0)),
        out_shape=jax.ShapeDtypeStruct((N, D), jnp.float32),
    )(h, parts0, parts0, parts1, parts1, g, b)


def _pool_body(h_ref, pw1_ref, pb1_ref, pw2_ref, pb2_ref, out_ref,
               sum_ref, max_ref):
    i = pl.program_id(0)

    @pl.when(i == 0)
    def _():
        sum_ref[...] = jnp.zeros_like(sum_ref)
        max_ref[...] = jnp.full_like(max_ref, -jnp.inf)

    blk = h_ref[...]
    sum_ref[...] += jnp.broadcast_to(jnp.sum(blk, axis=0, keepdims=True), (8, D))
    max_ref[...] = jnp.maximum(
        max_ref[...], jnp.broadcast_to(jnp.max(blk, axis=0, keepdims=True), (8, D)))

    @pl.when(i == N_BLOCKS - 1)
    def _():
        mean8 = sum_ref[...] * (1.0 / N)
        pin = jnp.concatenate([mean8, max_ref[...]], axis=-1)
        hdn = jax.nn.gelu(
            jnp.dot(pin, pw1_ref[...], preferred_element_type=jnp.float32)
            + pb1_ref[...])
        out_ref[...] = (
            jnp.dot(hdn, pw2_ref[...], preferred_element_type=jnp.float32)
            + pb2_ref[...])


def _pool_call(h, pw1, pb1, pw2, pb2):
    return pl.pallas_call(
        _pool_body,
        grid=(N_BLOCKS,),
        in_specs=[
            pl.BlockSpec((NB, D), lambda i: (i, 0)),
            pl.BlockSpec((2 * D, D), lambda i: (0, 0)),
            pl.BlockSpec((1, D), lambda i: (0, 0)),
            pl.BlockSpec((D, D), lambda i: (0, 0)),
            pl.BlockSpec((1, D), lambda i: (0, 0)),
        ],
        out_specs=pl.BlockSpec((8, D), lambda i: (0, 0)),
        out_shape=jax.ShapeDtypeStruct((8, D), jnp.float32),
        scratch_shapes=[
            pltpu.VMEM((8, D), jnp.float32),
            pltpu.VMEM((8, D), jnp.float32),
        ],
    )(h, pw1, pb1, pw2, pb2)


def kernel(node_states, edge_index, rel_ids, rel_emb,
           msg_W1, msg_b1, msg_W2, msg_b2,
           gate_W1, gate_b1, gate_W2, gate_b2,
           ln_g, ln_b, pool_W1, pool_b1, pool_W2, pool_b2):
    src = edge_index[0]
    dst = edge_index[1]
    rel8 = jnp.pad(rel_emb, ((0, 8 - rel_emb.shape[0]), (0, 0)))
    zeros_nd = jnp.zeros((NP, D), jnp.float32)
    eidx = _eidx_call(src.reshape(1, 1, E), rel_ids.reshape(1, 1, E)).reshape(E)

    bounds = [(0, E0), (E0, E)]
    chunks = []
    for lo, hi in bounds:
        eh = hi - lo
        epw = eh // NW
        chunks.append(dict(
            epw=epw,
            e3=eidx[lo:hi].reshape(NW, epw // GC, GC),
            d3=dst[lo:hi].reshape(NW, epw // GC, GC),
            rel3=rel_ids[lo:hi].reshape(eh // EB, 1, EB),
        ))

    h = node_states
    L = msg_W1.shape[0]
    for l in range(L):
        wall = jnp.concatenate(
            [msg_W1[l][:D], gate_W1[l][D:2 * D], gate_W1[l][:D]], axis=1)
        wrel = jnp.concatenate([msg_W1[l][D:], gate_W1[l][2 * D:]], axis=1)
        brel = jnp.concatenate([msg_b1[l], gate_b1[l]])[None, :]
        t, pgd, reltab = _proj_call(h, wall, rel8, wrel, brel,
                                    msg_W2[l].astype(jnp.bfloat16),
                                    msg_b2[l][None, :])
        parts = []
        for ck in chunks:
            gsrc, gdst = _gather_call(t, pgd, ck["e3"], ck["d3"], ck["epw"])
            ge = _edge_call(gsrc, gdst, ck["rel3"], reltab,
                            gate_W2[l].T, gate_b2[l][None, :])
            parts.append(_scatter_call(ge, ck["d3"], zeros_nd, ck["epw"]))
        h = _ln_call(h, parts[0], parts[1], ln_g[l][None, :], ln_b[l][None, :])

    pooled = _pool_call(h, pool_W1, pool_b1[None, :], pool_W2, pool_b2[None, :])
    return jnp.concatenate([h, pooled[:1]], axis=0)


# speedup vs baseline: 1.6204x; 1.0356x over previous
import jax
import jax.numpy as jnp
from jax import lax
from jax.experimental import pallas as pl
from jax.experimental.pallas import tpu as pltpu
from jax.experimental.pallas import tpu_sc as plsc

N = 10000
E = 320000
D = 128

NC = 2
NS = 16
NW = NC * NS

NB = 400
N_BLOCKS = N // NB


def _bf16_bits(x):
    b = lax.bitcast_convert_type(x, jnp.int32)
    b = b + jnp.int32(0x7FFF) + (lax.shift_right_logical(b, 16) & jnp.int32(1))
    return b & jnp.int32(-65536)


def _pack2(hi_f32, lo_f32):
    return _bf16_bits(hi_f32) | lax.shift_right_logical(_bf16_bits(lo_f32), 16)


def _unpack_hi(i32):
    return lax.bitcast_convert_type(i32 & jnp.int32(-65536), jnp.float32)


def _unpack_lo(i32):
    return lax.bitcast_convert_type(lax.shift_left(i32, 16), jnp.float32)


PB = 80
P_BLOCKS = N // PB
NR8 = N * 8


def _proj_body(h_ref, wall_ref, rel8_ref, wrel_ref, brel_ref, w2_ref, b2_ref,
               t_ref, pgd_ref, reltab_ref):
    h = h_ref[...]
    p = jnp.dot(h, wall_ref[...], preferred_element_type=jnp.float32)
    pm, pgs, pgd = p[:, :D], p[:, D:2 * D], p[:, 2 * D:]
    pgd_ref[...] = pgd
    rt = (jnp.dot(rel8_ref[...], wrel_ref[...], preferred_element_type=jnp.float32)
          + brel_ref[...])
    relm = rt[:, :D]
    u = jax.nn.gelu(pm[:, None, :] + relm[None, :, :]).reshape(PB * 8, D)
    m = (jnp.dot(u.astype(jnp.bfloat16), w2_ref[...],
                 preferred_element_type=jnp.float32) + b2_ref[...])
    pgs_b = jnp.broadcast_to(pgs[:, None, :], (PB, 8, D)).reshape(PB * 8, D)
    t_ref[...] = _pack2(m, pgs_b)

    @pl.when(pl.program_id(0) == 0)
    def _():
        reltab_ref[...] = rt


def _proj_call(h, wall, rel8, wrel, brel, w2, b2):
    return pl.pallas_call(
        _proj_body,
        grid=(P_BLOCKS,),
        in_specs=[
            pl.BlockSpec((PB, D), lambda i: (i, 0)),
            pl.BlockSpec((D, 3 * D), lambda i: (0, 0)),
            pl.BlockSpec((8, D), lambda i: (0, 0)),
            pl.BlockSpec((D, 2 * D), lambda i: (0, 0)),
            pl.BlockSpec((1, 2 * D), lambda i: (0, 0)),
            pl.BlockSpec((D, D), lambda i: (0, 0)),
            pl.BlockSpec((1, D), lambda i: (0, 0)),
        ],
        out_specs=[
            pl.BlockSpec((PB * 8, D), lambda i: (i, 0)),
            pl.BlockSpec((PB, D), lambda i: (i, 0)),
            pl.BlockSpec((8, 2 * D), lambda i: (0, 0)),
        ],
        out_shape=[
            jax.ShapeDtypeStruct((NR8, D), jnp.int32),
            jax.ShapeDtypeStruct((N, D), jnp.float32),
            jax.ShapeDtypeStruct((8, 2 * D), jnp.float32),
        ],
    )(h, wall, rel8, wrel, brel, w2, b2)


def _eidx_body(src_ref, rel_ref, out_ref):
    out_ref[...] = src_ref[...] * 8 + rel_ref[...]


def _eidx_call(src3, rel3flat):
    return pl.pallas_call(
        _eidx_body,
        out_shape=jax.ShapeDtypeStruct((1, 1, E), jnp.int32),
    )(src3, rel3flat)


GC = 80
E0 = 158720


_MASKHI = -65536


def _gather_call(t, pgd, src3, dst3, epw):
    niters = epw // GC
    eh = epw * NW
    mesh = plsc.VectorSubcoreMesh(core_axis_name="c", subcore_axis_name="s")

    def body(t_hbm, pgd_hbm, src3_hbm, dst3_hbm, gsrc_hbm, gdst_hbm,
             sidx, didx, sbufs, dbufs, semt, semp):
        wid = lax.axis_index("s") * NC + lax.axis_index("c")
        pltpu.sync_copy(src3_hbm.at[wid], sidx)
        pltpu.sync_copy(dst3_hbm.at[wid], didx)
        base = wid * epw

        def fire(it, p):
            pltpu.async_copy(t_hbm.at[sidx.at[it]], sbufs.at[p], semt)
            pltpu.async_copy(pgd_hbm.at[didx.at[it]], dbufs.at[p], semp)

        fire(0, 0)

        def step(it, carry):
            p = lax.rem(it, 2)

            @pl.when(it + 1 < niters)
            def _():
                fire(it + 1, 1 - p)

            pltpu.make_async_copy(t_hbm.at[sidx.at[it]], sbufs.at[p], semt).wait()
            pltpu.make_async_copy(pgd_hbm.at[didx.at[it]], dbufs.at[p], semp).wait()
            off = pl.multiple_of(base + it * GC, 8)
            pltpu.sync_copy(sbufs.at[p], gsrc_hbm.at[pl.ds(off, GC)])
            pltpu.sync_copy(dbufs.at[p], gdst_hbm.at[pl.ds(off, GC)])
            return carry

        lax.fori_loop(0, niters, step, 0)

    f = pl.kernel(
        body,
        out_type=[
            jax.ShapeDtypeStruct((eh, D), jnp.int32),
            jax.ShapeDtypeStruct((eh, D), jnp.float32),
        ],
        mesh=mesh,
        scratch_types=[
            pltpu.VMEM((niters, GC), jnp.int32),
            pltpu.VMEM((niters, GC), jnp.int32),
            pltpu.VMEM((2, GC, D), jnp.int32),
            pltpu.VMEM((2, GC, D), jnp.float32),
            pltpu.SemaphoreType.DMA,
            pltpu.SemaphoreType.DMA,
        ],
    )
    return f(t, pgd, src3, dst3)


EB = 512
E_BLOCKS = E // EB


def _edge_body(gsrc_ref, gdst_ref, rel_ref, reltab_ref,
               gw2_ref, gb2_ref, ge_ref):
    ids = rel_ref[0, 0, :]
    onehot = (ids[:, None] == lax.broadcasted_iota(jnp.int32, (EB, 8), 1)
              ).astype(jnp.float32)
    addend = jnp.dot(onehot, reltab_ref[...], preferred_element_type=jnp.float32)
    gi = gsrc_ref[...]
    m = _unpack_hi(gi)
    v = jax.nn.gelu(gdst_ref[...] + _unpack_lo(gi) + addend[:, D:])
    gsc = jnp.sum(v * gw2_ref[...], axis=-1, keepdims=True) + gb2_ref[...]
    ge_ref[...] = jax.nn.sigmoid(gsc) * m


def _edge_call(gsrc, gdst, rel3, reltab, gw2row, gb2):
    eh = gsrc.shape[0]
    return pl.pallas_call(
        _edge_body,
        grid=(eh // EB,),
        in_specs=[
            pl.BlockSpec((EB, D), lambda i: (i, 0)),
            pl.BlockSpec((EB, D), lambda i: (i, 0)),
            pl.BlockSpec((1, 1, EB), lambda i: (i, 0, 0)),
            pl.BlockSpec((8, 2 * D), lambda i: (0, 0)),
            pl.BlockSpec((1, D), lambda i: (0, 0)),
            pl.BlockSpec((1, 1), lambda i: (0, 0)),
        ],
        out_specs=pl.BlockSpec((EB, D), lambda i: (i, 0)),
        out_shape=jax.ShapeDtypeStruct((eh, D), jnp.float32),
    )(gsrc, gdst, rel3, reltab, gw2row, gb2)


NP = 10240
RPT = NP // NS


def _scatter_call(ge, dst3, zeros_nd, ept):
    niters = ept // GC
    mesh = plsc.VectorSubcoreMesh(core_axis_name="c", subcore_axis_name="s")

    def body(ge_hbm, dst3_hbm, zeros_hbm, parts_hbm, idx, rbufs, agg_sh, semr):
        c = lax.axis_index("c")
        s = lax.axis_index("s")
        wid = c * NS + s
        pltpu.sync_copy(zeros_hbm.at[pl.ds(s * RPT, RPT)],
                        agg_sh.at[pl.ds(s * RPT, RPT)])
        pltpu.sync_copy(dst3_hbm.at[wid], idx)
        base = wid * ept
        plsc.subcore_barrier()

        def fire(it, p):
            off = pl.multiple_of(base + it * GC, 8)
            pltpu.async_copy(ge_hbm.at[pl.ds(off, GC)], rbufs.at[p], semr)

        fire(0, 0)

        def step(it, carry):
            p = lax.rem(it, 2)

            @pl.when(it + 1 < niters)
            def _():
                fire(it + 1, 1 - p)

            pltpu.make_async_copy(ge_hbm.at[pl.ds(base, GC)],
                                  rbufs.at[p], semr).wait()
            pltpu.sync_copy(rbufs.at[p], agg_sh.at[idx.at[it]], add=True)
            return carry

        lax.fori_loop(0, niters, step, 0)
        plsc.subcore_barrier()
        pltpu.sync_copy(agg_sh.at[pl.ds(s * RPT, RPT)],
                        parts_hbm.at[c, pl.ds(s * RPT, RPT)])

    f = pl.kernel(
        body,
        out_type=jax.ShapeDtypeStruct((NC, NP, D), jnp.float32),
        mesh=mesh,
        scratch_types=[
            pltpu.VMEM((niters, GC), jnp.int32),
            pltpu.VMEM((2, GC, D), jnp.float32),
            pltpu.VMEM_SHARED((NP, D), jnp.float32),
            pltpu.SemaphoreType.DMA,
        ],
    )
    return f(ge, dst3, zeros_nd)


def _ln_call(h, parts_list, g, b):
    nparts = len(parts_list)

    def body(*refs):
        h_ref = refs[0]
        part_refs = refs[1:1 + 2 * nparts]
        g_ref, b_ref, out_ref = refs[1 + 2 * nparts:]
        x = h_ref[...]
        for pr in part_refs:
            x = x + pr[0]
        mu = jnp.mean(x, axis=-1, keepdims=True)
        xc = x - mu
        var = jnp.mean(xc * xc, axis=-1, keepdims=True)
        out_ref[...] = xc * lax.rsqrt(var + 1e-5) * g_ref[...] + b_ref[...]

    part_specs = []
    for _ in range(nparts):
        part_specs.append(pl.BlockSpec((1, NB, D), lambda i: (0, i, 0)))
        part_specs.append(pl.BlockSpec((1, NB, D), lambda i: (1, i, 0)))
    part_args = [p for p in parts_list for _ in range(2)]
    return pl.pallas_call(
        body,
        grid=(N_BLOCKS,),
        in_specs=(
            [pl.BlockSpec((NB, D), lambda i: (i, 0))]
            + part_specs
            + [pl.BlockSpec((1, D), lambda i: (0, 0)),
               pl.BlockSpec((1, D), lambda i: (0, 0))]
        ),
        out_specs=pl.BlockSpec((NB, D), lambda i: (i, 0)),
        out_shape=jax.ShapeDtypeStruct((N, D), jnp.float32),
    )(h, *part_args, g, b)


def _pool_body(h_ref, pw1_ref, pb1_ref, pw2_ref, pb2_ref, out_ref,
               sum_ref, max_ref):
    i = pl.program_id(0)

    @pl.when(i == 0)
    def _():
        sum_ref[...] = jnp.zeros_like(sum_ref)
        max_ref[...] = jnp.full_like(max_ref, -jnp.inf)

    blk = h_ref[...]
    sum_ref[...] += jnp.broadcast_to(jnp.sum(blk, axis=0, keepdims=True), (8, D))
    max_ref[...] = jnp.maximum(
        max_ref[...], jnp.broadcast_to(jnp.max(blk, axis=0, keepdims=True), (8, D)))

    @pl.when(i == N_BLOCKS - 1)
    def _():
        mean8 = sum_ref[...] * (1.0 / N)
        pin = jnp.concatenate([mean8, max_ref[...]], axis=-1)
        hdn = jax.nn.gelu(
            jnp.dot(pin, pw1_ref[...], preferred_element_type=jnp.float32)
            + pb1_ref[...])
        out_ref[...] = (
            jnp.dot(hdn, pw2_ref[...], preferred_element_type=jnp.float32)
            + pb2_ref[...])


def _pool_call(h, pw1, pb1, pw2, pb2):
    return pl.pallas_call(
        _pool_body,
        grid=(N_BLOCKS,),
        in_specs=[
            pl.BlockSpec((NB, D), lambda i: (i, 0)),
            pl.BlockSpec((2 * D, D), lambda i: (0, 0)),
            pl.BlockSpec((1, D), lambda i: (0, 0)),
            pl.BlockSpec((D, D), lambda i: (0, 0)),
            pl.BlockSpec((1, D), lambda i: (0, 0)),
        ],
        out_specs=pl.BlockSpec((8, D), lambda i: (0, 0)),
        out_shape=jax.ShapeDtypeStruct((8, D), jnp.float32),
        scratch_shapes=[
            pltpu.VMEM((8, D), jnp.float32),
            pltpu.VMEM((8, D), jnp.float32),
        ],
    )(h, pw1, pb1, pw2, pb2)


def kernel(node_states, edge_index, rel_ids, rel_emb,
           msg_W1, msg_b1, msg_W2, msg_b2,
           gate_W1, gate_b1, gate_W2, gate_b2,
           ln_g, ln_b, pool_W1, pool_b1, pool_W2, pool_b2):
    src = edge_index[0]
    dst = edge_index[1]
    rel8 = jnp.pad(rel_emb, ((0, 8 - rel_emb.shape[0]), (0, 0)))
    zeros_nd = jnp.zeros((NP, D), jnp.float32)
    eidx = _eidx_call(src.reshape(1, 1, E), rel_ids.reshape(1, 1, E)).reshape(E)

    unit = NW * GC
    nchunks = min(4, E // unit)
    tu = E // unit
    cuts = [round(i * tu / nchunks) * unit for i in range(nchunks + 1)]
    bounds = list(zip(cuts[:-1], cuts[1:]))
    chunks = []
    for lo, hi in bounds:
        eh = hi - lo
        epw = eh // NW
        chunks.append(dict(
            epw=epw,
            e3=eidx[lo:hi].reshape(NW, epw // GC, GC),
            d3=dst[lo:hi].reshape(NW, epw // GC, GC),
            rel3=rel_ids[lo:hi].reshape(eh // EB, 1, EB),
        ))

    h = node_states
    L = msg_W1.shape[0]
    for l in range(L):
        wall = jnp.concatenate(
            [msg_W1[l][:D], gate_W1[l][D:2 * D], gate_W1[l][:D]], axis=1)
        wrel = jnp.concatenate([msg_W1[l][D:], gate_W1[l][2 * D:]], axis=1)
        brel = jnp.concatenate([msg_b1[l], gate_b1[l]])[None, :]
        t, pgd, reltab = _proj_call(h, wall, rel8, wrel, brel,
                                    msg_W2[l].astype(jnp.bfloat16),
                                    msg_b2[l][None, :])
        parts = []
        for ck in chunks:
            gsrc, gdst = _gather_call(t, pgd, ck["e3"], ck["d3"], ck["epw"])
            ge = _edge_call(gsrc, gdst, ck["rel3"], reltab,
                            gate_W2[l].T, gate_b2[l][None, :])
            parts.append(_scatter_call(ge, ck["d3"], zeros_nd, ck["epw"]))
        h = _ln_call(h, parts, ln_g[l][None, :], ln_b[l][None, :])

    pooled = _pool_call(h, pool_W1, pool_b1[None, :], pool_W2, pool_b2[None, :])
    return jnp.concatenate([h, pooled[:1]], axis=0)


# speedup vs baseline: 1.6782x; 1.0357x over previous
import jax
import jax.numpy as jnp
from jax import lax
from jax.experimental import pallas as pl
from jax.experimental.pallas import tpu as pltpu
from jax.experimental.pallas import tpu_sc as plsc

N = 10000
E = 320000
D = 128

NC = 2
NS = 16
NW = NC * NS

NB = 400
N_BLOCKS = N // NB


def _bf16_bits(x):
    b = lax.bitcast_convert_type(x, jnp.int32)
    b = b + jnp.int32(0x7FFF) + (lax.shift_right_logical(b, 16) & jnp.int32(1))
    return b & jnp.int32(-65536)


def _pack2(hi_f32, lo_f32):
    return _bf16_bits(hi_f32) | lax.shift_right_logical(_bf16_bits(lo_f32), 16)


def _unpack_hi(i32):
    return lax.bitcast_convert_type(i32 & jnp.int32(-65536), jnp.float32)


def _unpack_lo(i32):
    return lax.bitcast_convert_type(lax.shift_left(i32, 16), jnp.float32)


PB = 80
P_BLOCKS = N // PB
NR8 = N * 8


def _proj_body(h_ref, wall_ref, rel8_ref, wrel_ref, brel_ref, w2_ref, b2_ref,
               t_ref, pgd_ref, reltab_ref):
    h = h_ref[...]
    p = jnp.dot(h, wall_ref[...], preferred_element_type=jnp.float32)
    pm, pgs, pgd = p[:, :D], p[:, D:2 * D], p[:, 2 * D:]
    pgd_ref[...] = pgd
    rt = (jnp.dot(rel8_ref[...], wrel_ref[...], preferred_element_type=jnp.float32)
          + brel_ref[...])
    relm = rt[:, :D]
    u = jax.nn.gelu(pm[:, None, :] + relm[None, :, :]).reshape(PB * 8, D)
    m = (jnp.dot(u.astype(jnp.bfloat16), w2_ref[...],
                 preferred_element_type=jnp.float32) + b2_ref[...])
    pgs_b = jnp.broadcast_to(pgs[:, None, :], (PB, 8, D)).reshape(PB * 8, D)
    t_ref[...] = _pack2(m, pgs_b)

    @pl.when(pl.program_id(0) == 0)
    def _():
        reltab_ref[...] = rt


def _proj_call(h, wall, rel8, wrel, brel, w2, b2):
    return pl.pallas_call(
        _proj_body,
        grid=(P_BLOCKS,),
        in_specs=[
            pl.BlockSpec((PB, D), lambda i: (i, 0)),
            pl.BlockSpec((D, 3 * D), lambda i: (0, 0)),
            pl.BlockSpec((8, D), lambda i: (0, 0)),
            pl.BlockSpec((D, 2 * D), lambda i: (0, 0)),
            pl.BlockSpec((1, 2 * D), lambda i: (0, 0)),
            pl.BlockSpec((D, D), lambda i: (0, 0)),
            pl.BlockSpec((1, D), lambda i: (0, 0)),
        ],
        out_specs=[
            pl.BlockSpec((PB * 8, D), lambda i: (i, 0)),
            pl.BlockSpec((PB, D), lambda i: (i, 0)),
            pl.BlockSpec((8, 2 * D), lambda i: (0, 0)),
        ],
        out_shape=[
            jax.ShapeDtypeStruct((NR8, D), jnp.int32),
            jax.ShapeDtypeStruct((N, D), jnp.float32),
            jax.ShapeDtypeStruct((8, 2 * D), jnp.float32),
        ],
    )(h, wall, rel8, wrel, brel, w2, b2)


def _eidx_body(src_ref, rel_ref, out_ref):
    out_ref[...] = src_ref[...] * 8 + rel_ref[...]


def _eidx_call(src3, rel3flat):
    return pl.pallas_call(
        _eidx_body,
        out_shape=jax.ShapeDtypeStruct((1, 1, E), jnp.int32),
    )(src3, rel3flat)


GC = 80
E0 = 158720


_MASKHI = -65536


def _gather_call(t, pgd, src3, dst3, epw):
    niters = epw // GC
    eh = epw * NW
    mesh = plsc.VectorSubcoreMesh(core_axis_name="c", subcore_axis_name="s")

    def body(t_hbm, pgd_hbm, src3_hbm, dst3_hbm, gsrc_hbm, gdst_hbm,
             sidx, didx, sbufs, dbufs, pgd_sh, semt, semp):
        sid = lax.axis_index("s")
        wid = sid * NC + lax.axis_index("c")
        rstage = (N // NS) // 8 * 8
        rlast = N - (NS - 1) * rstage

        @pl.when(sid < NS - 1)
        def _():
            off = pl.multiple_of(sid * rstage, 8)
            pltpu.sync_copy(pgd_hbm.at[pl.ds(off, rstage)],
                            pgd_sh.at[pl.ds(off, rstage)])

        @pl.when(sid == NS - 1)
        def _():
            pltpu.sync_copy(pgd_hbm.at[pl.ds((NS - 1) * rstage, rlast)],
                            pgd_sh.at[pl.ds((NS - 1) * rstage, rlast)])

        pltpu.sync_copy(src3_hbm.at[wid], sidx)
        pltpu.sync_copy(dst3_hbm.at[wid], didx)
        base = wid * epw
        plsc.subcore_barrier()

        def fire(it, p):
            pltpu.async_copy(t_hbm.at[sidx.at[it]], sbufs.at[p], semt)
            pltpu.async_copy(pgd_sh.at[didx.at[it]], dbufs.at[p], semp)

        fire(0, 0)

        def step(it, carry):
            p = lax.rem(it, 2)

            @pl.when(it + 1 < niters)
            def _():
                fire(it + 1, 1 - p)

            pltpu.make_async_copy(t_hbm.at[sidx.at[it]], sbufs.at[p], semt).wait()
            pltpu.make_async_copy(pgd_sh.at[didx.at[it]], dbufs.at[p], semp).wait()
            off = pl.multiple_of(base + it * GC, 8)
            pltpu.sync_copy(sbufs.at[p], gsrc_hbm.at[pl.ds(off, GC)])
            pltpu.sync_copy(dbufs.at[p], gdst_hbm.at[pl.ds(off, GC)])
            return carry

        lax.fori_loop(0, niters, step, 0)

    f = pl.kernel(
        body,
        out_type=[
            jax.ShapeDtypeStruct((eh, D), jnp.int32),
            jax.ShapeDtypeStruct((eh, D), jnp.float32),
        ],
        mesh=mesh,
        scratch_types=[
            pltpu.VMEM((niters, GC), jnp.int32),
            pltpu.VMEM((niters, GC), jnp.int32),
            pltpu.VMEM((2, GC, D), jnp.int32),
            pltpu.VMEM((2, GC, D), jnp.float32),
            pltpu.VMEM_SHARED((N, D), jnp.float32),
            pltpu.SemaphoreType.DMA,
            pltpu.SemaphoreType.DMA,
        ],
    )
    return f(t, pgd, src3, dst3)


EB = 512
E_BLOCKS = E // EB


def _edge_body(gsrc_ref, gdst_ref, rel_ref, reltab_ref,
               gw2_ref, gb2_ref, ge_ref):
    ids = rel_ref[0, 0, :]
    onehot = (ids[:, None] == lax.broadcasted_iota(jnp.int32, (EB, 8), 1)
              ).astype(jnp.float32)
    addend = jnp.dot(onehot, reltab_ref[...], preferred_element_type=jnp.float32)
    gi = gsrc_ref[...]
    m = _unpack_hi(gi)
    v = jax.nn.gelu(gdst_ref[...] + _unpack_lo(gi) + addend[:, D:])
    gsc = jnp.sum(v * gw2_ref[...], axis=-1, keepdims=True) + gb2_ref[...]
    ge_ref[...] = jax.nn.sigmoid(gsc) * m


def _edge_call(gsrc, gdst, rel3, reltab, gw2row, gb2):
    eh = gsrc.shape[0]
    return pl.pallas_call(
        _edge_body,
        grid=(eh // EB,),
        in_specs=[
            pl.BlockSpec((EB, D), lambda i: (i, 0)),
            pl.BlockSpec((EB, D), lambda i: (i, 0)),
            pl.BlockSpec((1, 1, EB), lambda i: (i, 0, 0)),
            pl.BlockSpec((8, 2 * D), lambda i: (0, 0)),
            pl.BlockSpec((1, D), lambda i: (0, 0)),
            pl.BlockSpec((1, 1), lambda i: (0, 0)),
        ],
        out_specs=pl.BlockSpec((EB, D), lambda i: (i, 0)),
        out_shape=jax.ShapeDtypeStruct((eh, D), jnp.float32),
    )(gsrc, gdst, rel3, reltab, gw2row, gb2)


NP = 10240
RPT = NP // NS


def _scatter_call(ge, dst3, zeros_nd, ept):
    niters = ept // GC
    mesh = plsc.VectorSubcoreMesh(core_axis_name="c", subcore_axis_name="s")

    def body(ge_hbm, dst3_hbm, zeros_hbm, parts_hbm, idx, rbufs, agg_sh, semr):
        c = lax.axis_index("c")
        s = lax.axis_index("s")
        wid = c * NS + s
        pltpu.sync_copy(zeros_hbm.at[pl.ds(s * RPT, RPT)],
                        agg_sh.at[pl.ds(s * RPT, RPT)])
        pltpu.sync_copy(dst3_hbm.at[wid], idx)
        base = wid * ept
        plsc.subcore_barrier()

        def fire(it, p):
            off = pl.multiple_of(base + it * GC, 8)
            pltpu.async_copy(ge_hbm.at[pl.ds(off, GC)], rbufs.at[p], semr)

        fire(0, 0)

        def step(it, carry):
            p = lax.rem(it, 2)

            @pl.when(it + 1 < niters)
            def _():
                fire(it + 1, 1 - p)

            pltpu.make_async_copy(ge_hbm.at[pl.ds(base, GC)],
                                  rbufs.at[p], semr).wait()
            pltpu.sync_copy(rbufs.at[p], agg_sh.at[idx.at[it]], add=True)
            return carry

        lax.fori_loop(0, niters, step, 0)
        plsc.subcore_barrier()
        pltpu.sync_copy(agg_sh.at[pl.ds(s * RPT, RPT)],
                        parts_hbm.at[c, pl.ds(s * RPT, RPT)])

    f = pl.kernel(
        body,
        out_type=jax.ShapeDtypeStruct((NC, NP, D), jnp.float32),
        mesh=mesh,
        scratch_types=[
            pltpu.VMEM((niters, GC), jnp.int32),
            pltpu.VMEM((2, GC, D), jnp.float32),
            pltpu.VMEM_SHARED((NP, D), jnp.float32),
            pltpu.SemaphoreType.DMA,
        ],
    )
    return f(ge, dst3, zeros_nd)


def _ln_call(h, parts_list, g, b):
    nparts = len(parts_list)

    def body(*refs):
        h_ref = refs[0]
        part_refs = refs[1:1 + 2 * nparts]
        g_ref, b_ref, out_ref = refs[1 + 2 * nparts:]
        x = h_ref[...]
        for pr in part_refs:
            x = x + pr[0]
        mu = jnp.mean(x, axis=-1, keepdims=True)
        xc = x - mu
        var = jnp.mean(xc * xc, axis=-1, keepdims=True)
        out_ref[...] = xc * lax.rsqrt(var + 1e-5) * g_ref[...] + b_ref[...]

    part_specs = []
    for _ in range(nparts):
        part_specs.append(pl.BlockSpec((1, NB, D), lambda i: (0, i, 0)))
        part_specs.append(pl.BlockSpec((1, NB, D), lambda i: (1, i, 0)))
    part_args = [p for p in parts_list for _ in range(2)]
    return pl.pallas_call(
        body,
        grid=(N_BLOCKS,),
        in_specs=(
            [pl.BlockSpec((NB, D), lambda i: (i, 0))]
            + part_specs
            + [pl.BlockSpec((1, D), lambda i: (0, 0)),
               pl.BlockSpec((1, D), lambda i: (0, 0))]
        ),
        out_specs=pl.BlockSpec((NB, D), lambda i: (i, 0)),
        out_shape=jax.ShapeDtypeStruct((N, D), jnp.float32),
    )(h, *part_args, g, b)


def _pool_body(h_ref, pw1_ref, pb1_ref, pw2_ref, pb2_ref, out_ref,
               sum_ref, max_ref):
    i = pl.program_id(0)

    @pl.when(i == 0)
    def _():
        sum_ref[...] = jnp.zeros_like(sum_ref)
        max_ref[...] = jnp.full_like(max_ref, -jnp.inf)

    blk = h_ref[...]
    sum_ref[...] += jnp.broadcast_to(jnp.sum(blk, axis=0, keepdims=True), (8, D))
    max_ref[...] = jnp.maximum(
        max_ref[...], jnp.broadcast_to(jnp.max(blk, axis=0, keepdims=True), (8, D)))

    @pl.when(i == N_BLOCKS - 1)
    def _():
        mean8 = sum_ref[...] * (1.0 / N)
        pin = jnp.concatenate([mean8, max_ref[...]], axis=-1)
        hdn = jax.nn.gelu(
            jnp.dot(pin, pw1_ref[...], preferred_element_type=jnp.float32)
            + pb1_ref[...])
        out_ref[...] = (
            jnp.dot(hdn, pw2_ref[...], preferred_element_type=jnp.float32)
            + pb2_ref[...])


def _pool_call(h, pw1, pb1, pw2, pb2):
    return pl.pallas_call(
        _pool_body,
        grid=(N_BLOCKS,),
        in_specs=[
            pl.BlockSpec((NB, D), lambda i: (i, 0)),
            pl.BlockSpec((2 * D, D), lambda i: (0, 0)),
            pl.BlockSpec((1, D), lambda i: (0, 0)),
            pl.BlockSpec((D, D), lambda i: (0, 0)),
            pl.BlockSpec((1, D), lambda i: (0, 0)),
        ],
        out_specs=pl.BlockSpec((8, D), lambda i: (0, 0)),
        out_shape=jax.ShapeDtypeStruct((8, D), jnp.float32),
        scratch_shapes=[
            pltpu.VMEM((8, D), jnp.float32),
            pltpu.VMEM((8, D), jnp.float32),
        ],
    )(h, pw1, pb1, pw2, pb2)


def kernel(node_states, edge_index, rel_ids, rel_emb,
           msg_W1, msg_b1, msg_W2, msg_b2,
           gate_W1, gate_b1, gate_W2, gate_b2,
           ln_g, ln_b, pool_W1, pool_b1, pool_W2, pool_b2):
    src = edge_index[0]
    dst = edge_index[1]
    rel8 = jnp.pad(rel_emb, ((0, 8 - rel_emb.shape[0]), (0, 0)))
    zeros_nd = jnp.zeros((NP, D), jnp.float32)
    eidx = _eidx_call(src.reshape(1, 1, E), rel_ids.reshape(1, 1, E)).reshape(E)

    unit = NW * GC
    nchunks = min(4, E // unit)
    tu = E // unit
    cuts = [round(i * tu / nchunks) * unit for i in range(nchunks + 1)]
    bounds = list(zip(cuts[:-1], cuts[1:]))
    chunks = []
    for lo, hi in bounds:
        eh = hi - lo
        epw = eh // NW
        chunks.append(dict(
            epw=epw,
            e3=eidx[lo:hi].reshape(NW, epw // GC, GC),
            d3=dst[lo:hi].reshape(NW, epw // GC, GC),
            rel3=rel_ids[lo:hi].reshape(eh // EB, 1, EB),
        ))

    h = node_states
    L = msg_W1.shape[0]
    for l in range(L):
        wall = jnp.concatenate(
            [msg_W1[l][:D], gate_W1[l][D:2 * D], gate_W1[l][:D]], axis=1)
        wrel = jnp.concatenate([msg_W1[l][D:], gate_W1[l][2 * D:]], axis=1)
        brel = jnp.concatenate([msg_b1[l], gate_b1[l]])[None, :]
        t, pgd, reltab = _proj_call(h, wall, rel8, wrel, brel,
                                    msg_W2[l].astype(jnp.bfloat16),
                                    msg_b2[l][None, :])
        parts = []
        for ck in chunks:
            gsrc, gdst = _gather_call(t, pgd, ck["e3"], ck["d3"], ck["epw"])
            ge = _edge_call(gsrc, gdst, ck["rel3"], reltab,
                            gate_W2[l].T, gate_b2[l][None, :])
            parts.append(_scatter_call(ge, ck["d3"], zeros_nd, ck["epw"]))
        h = _ln_call(h, parts, ln_g[l][None, :], ln_b[l][None, :])

    pooled = _pool_call(h, pool_W1, pool_b1[None, :], pool_W2, pool_b2[None, :])
    return jnp.concatenate([h, pooled[:1]], axis=0)


# speedup vs baseline: 1.6802x; 1.0012x over previous
import jax
import jax.numpy as jnp
from jax import lax
from jax.experimental import pallas as pl
from jax.experimental.pallas import tpu as pltpu
from jax.experimental.pallas import tpu_sc as plsc

N = 10000
E = 320000
D = 128

NC = 2
NS = 16
NW = NC * NS

NB = 400
N_BLOCKS = N // NB


def _bf16_bits(x):
    b = lax.bitcast_convert_type(x, jnp.int32)
    b = b + jnp.int32(0x7FFF) + (lax.shift_right_logical(b, 16) & jnp.int32(1))
    return b & jnp.int32(-65536)


def _pack2(hi_f32, lo_f32):
    return _bf16_bits(hi_f32) | lax.shift_right_logical(_bf16_bits(lo_f32), 16)


def _unpack_hi(i32):
    return lax.bitcast_convert_type(i32 & jnp.int32(-65536), jnp.float32)


def _unpack_lo(i32):
    return lax.bitcast_convert_type(lax.shift_left(i32, 16), jnp.float32)


PB = 80
P_BLOCKS = N // PB
NR8 = N * 8


def _proj_body(h_ref, wall_ref, rel8_ref, wrel_ref, brel_ref, w2_ref, b2_ref,
               t_ref, pgd_ref, reltab_ref):
    h = h_ref[...]
    p = jnp.dot(h, wall_ref[...], preferred_element_type=jnp.float32)
    pm, pgs, pgd = p[:, :D], p[:, D:2 * D], p[:, 2 * D:]
    pgd_ref[...] = pgd
    rt = (jnp.dot(rel8_ref[...], wrel_ref[...], preferred_element_type=jnp.float32)
          + brel_ref[...])
    relm = rt[:, :D]
    u = jax.nn.gelu(pm[:, None, :] + relm[None, :, :]).reshape(PB * 8, D)
    m = (jnp.dot(u.astype(jnp.bfloat16), w2_ref[...],
                 preferred_element_type=jnp.float32) + b2_ref[...])
    pgs_b = jnp.broadcast_to(pgs[:, None, :], (PB, 8, D)).reshape(PB * 8, D)
    t_ref[...] = _pack2(m, pgs_b)

    @pl.when(pl.program_id(0) == 0)
    def _():
        reltab_ref[...] = rt


def _proj_call(h, wall, rel8, wrel, brel, w2, b2):
    return pl.pallas_call(
        _proj_body,
        grid=(P_BLOCKS,),
        in_specs=[
            pl.BlockSpec((PB, D), lambda i: (i, 0)),
            pl.BlockSpec((D, 3 * D), lambda i: (0, 0)),
            pl.BlockSpec((8, D), lambda i: (0, 0)),
            pl.BlockSpec((D, 2 * D), lambda i: (0, 0)),
            pl.BlockSpec((1, 2 * D), lambda i: (0, 0)),
            pl.BlockSpec((D, D), lambda i: (0, 0)),
            pl.BlockSpec((1, D), lambda i: (0, 0)),
        ],
        out_specs=[
            pl.BlockSpec((PB * 8, D), lambda i: (i, 0)),
            pl.BlockSpec((PB, D), lambda i: (i, 0)),
            pl.BlockSpec((8, 2 * D), lambda i: (0, 0)),
        ],
        out_shape=[
            jax.ShapeDtypeStruct((NR8, D), jnp.int32),
            jax.ShapeDtypeStruct((N, D), jnp.float32),
            jax.ShapeDtypeStruct((8, 2 * D), jnp.float32),
        ],
    )(h, wall, rel8, wrel, brel, w2, b2)


def _eidx_body(src_ref, rel_ref, out_ref):
    out_ref[...] = src_ref[...] * 8 + rel_ref[...]


def _eidx_call(src3, rel3flat):
    return pl.pallas_call(
        _eidx_body,
        out_shape=jax.ShapeDtypeStruct((1, 1, E), jnp.int32),
    )(src3, rel3flat)


GC = 80
E0 = 158720


_MASKHI = -65536


def _gather_call(t, pgd, src3, dst3, epw):
    niters = epw // GC
    eh = epw * NW
    mesh = plsc.VectorSubcoreMesh(core_axis_name="c", subcore_axis_name="s")

    def body(t_hbm, pgd_hbm, src3_hbm, dst3_hbm, gsrc_hbm, gdst_hbm,
             sidx, didx, sbufs, dbufs, pgd_sh, semt, semp):
        sid = lax.axis_index("s")
        wid = sid * NC + lax.axis_index("c")
        rstage = (N // NS) // 8 * 8
        rlast = N - (NS - 1) * rstage

        @pl.when(sid < NS - 1)
        def _():
            off = pl.multiple_of(sid * rstage, 8)
            pltpu.sync_copy(pgd_hbm.at[pl.ds(off, rstage)],
                            pgd_sh.at[pl.ds(off, rstage)])

        @pl.when(sid == NS - 1)
        def _():
            pltpu.sync_copy(pgd_hbm.at[pl.ds((NS - 1) * rstage, rlast)],
                            pgd_sh.at[pl.ds((NS - 1) * rstage, rlast)])

        pltpu.sync_copy(src3_hbm.at[wid], sidx)
        pltpu.sync_copy(dst3_hbm.at[wid], didx)
        base = wid * epw
        plsc.subcore_barrier()

        def fire(it, p):
            pltpu.async_copy(t_hbm.at[sidx.at[it]], sbufs.at[p], semt)
            pltpu.async_copy(pgd_sh.at[didx.at[it]], dbufs.at[p], semp)

        fire(0, 0)

        def step(it, carry):
            p = lax.rem(it, 2)

            @pl.when(it + 1 < niters)
            def _():
                fire(it + 1, 1 - p)

            pltpu.make_async_copy(t_hbm.at[sidx.at[it]], sbufs.at[p], semt).wait()
            pltpu.make_async_copy(pgd_sh.at[didx.at[it]], dbufs.at[p], semp).wait()
            off = pl.multiple_of(base + it * GC, 8)
            pltpu.sync_copy(sbufs.at[p], gsrc_hbm.at[pl.ds(off, GC)])
            pltpu.sync_copy(dbufs.at[p], gdst_hbm.at[pl.ds(off, GC)])
            return carry

        lax.fori_loop(0, niters, step, 0)

    f = pl.kernel(
        body,
        out_type=[
            jax.ShapeDtypeStruct((eh, D), jnp.int32),
            jax.ShapeDtypeStruct((eh, D), jnp.float32),
        ],
        mesh=mesh,
        scratch_types=[
            pltpu.VMEM((niters, GC), jnp.int32),
            pltpu.VMEM((niters, GC), jnp.int32),
            pltpu.VMEM((2, GC, D), jnp.int32),
            pltpu.VMEM((2, GC, D), jnp.float32),
            pltpu.VMEM_SHARED((N, D), jnp.float32),
            pltpu.SemaphoreType.DMA,
            pltpu.SemaphoreType.DMA,
        ],
    )
    return f(t, pgd, src3, dst3)


EB = 512
E_BLOCKS = E // EB


def _edge_body(gsrc_ref, gdst_ref, rel_ref, reltab_ref,
               gw2_ref, gb2_ref, ge_ref):
    ids = rel_ref[0, 0, :]
    onehot = (ids[:, None] == lax.broadcasted_iota(jnp.int32, (EB, 8), 1)
              ).astype(jnp.float32)
    addend = jnp.dot(onehot, reltab_ref[...], preferred_element_type=jnp.float32)
    gi = gsrc_ref[...]
    m = _unpack_hi(gi)
    v = jax.nn.gelu(gdst_ref[...] + _unpack_lo(gi) + addend[:, D:])
    gsc = jnp.sum(v * gw2_ref[...], axis=-1, keepdims=True) + gb2_ref[...]
    ge_ref[...] = jax.nn.sigmoid(gsc) * m


def _edge_call(gsrc, gdst, rel3, reltab, gw2row, gb2):
    eh = gsrc.shape[0]
    return pl.pallas_call(
        _edge_body,
        grid=(eh // EB,),
        in_specs=[
            pl.BlockSpec((EB, D), lambda i: (i, 0)),
            pl.BlockSpec((EB, D), lambda i: (i, 0)),
            pl.BlockSpec((1, 1, EB), lambda i: (i, 0, 0)),
            pl.BlockSpec((8, 2 * D), lambda i: (0, 0)),
            pl.BlockSpec((1, D), lambda i: (0, 0)),
            pl.BlockSpec((1, 1), lambda i: (0, 0)),
        ],
        out_specs=pl.BlockSpec((EB, D), lambda i: (i, 0)),
        out_shape=jax.ShapeDtypeStruct((eh, D), jnp.float32),
    )(gsrc, gdst, rel3, reltab, gw2row, gb2)


NP = 10240
RPT = NP // NS


def _scatter_call(ge, dst3, zeros_nd, ept):
    niters = ept // GC
    mesh = plsc.VectorSubcoreMesh(core_axis_name="c", subcore_axis_name="s")

    def body(ge_hbm, dst3_hbm, zeros_hbm, parts_hbm, idx, rbufs, agg_sh, semr):
        c = lax.axis_index("c")
        s = lax.axis_index("s")
        wid = c * NS + s
        pltpu.sync_copy(zeros_hbm.at[pl.ds(s * RPT, RPT)],
                        agg_sh.at[pl.ds(s * RPT, RPT)])
        pltpu.sync_copy(dst3_hbm.at[wid], idx)
        base = wid * ept
        plsc.subcore_barrier()

        def fire(it, p):
            off = pl.multiple_of(base + it * GC, 8)
            pltpu.async_copy(ge_hbm.at[pl.ds(off, GC)], rbufs.at[p], semr)

        fire(0, 0)

        def step(it, carry):
            p = lax.rem(it, 2)

            @pl.when(it + 1 < niters)
            def _():
                fire(it + 1, 1 - p)

            pltpu.make_async_copy(ge_hbm.at[pl.ds(base, GC)],
                                  rbufs.at[p], semr).wait()
            pltpu.sync_copy(rbufs.at[p], agg_sh.at[idx.at[it]], add=True)
            return carry

        lax.fori_loop(0, niters, step, 0)
        plsc.subcore_barrier()
        pltpu.sync_copy(agg_sh.at[pl.ds(s * RPT, RPT)],
                        parts_hbm.at[c, pl.ds(s * RPT, RPT)])

    f = pl.kernel(
        body,
        out_type=jax.ShapeDtypeStruct((NC, NP, D), jnp.float32),
        mesh=mesh,
        scratch_types=[
            pltpu.VMEM((niters, GC), jnp.int32),
            pltpu.VMEM((2, GC, D), jnp.float32),
            pltpu.VMEM_SHARED((NP, D), jnp.float32),
            pltpu.SemaphoreType.DMA,
        ],
    )
    return f(ge, dst3, zeros_nd)


def _ln_call(h, parts_list, g, b):
    nparts = len(parts_list)

    def body(*refs):
        h_ref = refs[0]
        part_refs = refs[1:1 + 2 * nparts]
        g_ref, b_ref, out_ref = refs[1 + 2 * nparts:]
        x = h_ref[...]
        for pr in part_refs:
            x = x + pr[0]
        mu = jnp.mean(x, axis=-1, keepdims=True)
        xc = x - mu
        var = jnp.mean(xc * xc, axis=-1, keepdims=True)
        out_ref[...] = xc * lax.rsqrt(var + 1e-5) * g_ref[...] + b_ref[...]

    part_specs = []
    for _ in range(nparts):
        part_specs.append(pl.BlockSpec((1, NB, D), lambda i: (0, i, 0)))
        part_specs.append(pl.BlockSpec((1, NB, D), lambda i: (1, i, 0)))
    part_args = [p for p in parts_list for _ in range(2)]
    return pl.pallas_call(
        body,
        grid=(N_BLOCKS,),
        in_specs=(
            [pl.BlockSpec((NB, D), lambda i: (i, 0))]
            + part_specs
            + [pl.BlockSpec((1, D), lambda i: (0, 0)),
               pl.BlockSpec((1, D), lambda i: (0, 0))]
        ),
        out_specs=pl.BlockSpec((NB, D), lambda i: (i, 0)),
        out_shape=jax.ShapeDtypeStruct((N, D), jnp.float32),
    )(h, *part_args, g, b)


def _pool_body(h_ref, pw1_ref, pb1_ref, pw2_ref, pb2_ref, out_ref,
               sum_ref, max_ref):
    i = pl.program_id(0)

    @pl.when(i == 0)
    def _():
        sum_ref[...] = jnp.zeros_like(sum_ref)
        max_ref[...] = jnp.full_like(max_ref, -jnp.inf)

    blk = h_ref[...]
    sum_ref[...] += jnp.broadcast_to(jnp.sum(blk, axis=0, keepdims=True), (8, D))
    max_ref[...] = jnp.maximum(
        max_ref[...], jnp.broadcast_to(jnp.max(blk, axis=0, keepdims=True), (8, D)))

    @pl.when(i == N_BLOCKS - 1)
    def _():
        mean8 = sum_ref[...] * (1.0 / N)
        pin = jnp.concatenate([mean8, max_ref[...]], axis=-1)
        hdn = jax.nn.gelu(
            jnp.dot(pin, pw1_ref[...], preferred_element_type=jnp.float32)
            + pb1_ref[...])
        out_ref[...] = (
            jnp.dot(hdn, pw2_ref[...], preferred_element_type=jnp.float32)
            + pb2_ref[...])


def _pool_call(h, pw1, pb1, pw2, pb2):
    return pl.pallas_call(
        _pool_body,
        grid=(N_BLOCKS,),
        in_specs=[
            pl.BlockSpec((NB, D), lambda i: (i, 0)),
            pl.BlockSpec((2 * D, D), lambda i: (0, 0)),
            pl.BlockSpec((1, D), lambda i: (0, 0)),
            pl.BlockSpec((D, D), lambda i: (0, 0)),
            pl.BlockSpec((1, D), lambda i: (0, 0)),
        ],
        out_specs=pl.BlockSpec((8, D), lambda i: (0, 0)),
        out_shape=jax.ShapeDtypeStruct((8, D), jnp.float32),
        scratch_shapes=[
            pltpu.VMEM((8, D), jnp.float32),
            pltpu.VMEM((8, D), jnp.float32),
        ],
    )(h, pw1, pb1, pw2, pb2)


def kernel(node_states, edge_index, rel_ids, rel_emb,
           msg_W1, msg_b1, msg_W2, msg_b2,
           gate_W1, gate_b1, gate_W2, gate_b2,
           ln_g, ln_b, pool_W1, pool_b1, pool_W2, pool_b2):
    src = edge_index[0]
    dst = edge_index[1]
    rel8 = jnp.pad(rel_emb, ((0, 8 - rel_emb.shape[0]), (0, 0)))
    zeros_nd = jnp.zeros((NP, D), jnp.float32)
    eidx = _eidx_call(src.reshape(1, 1, E), rel_ids.reshape(1, 1, E)).reshape(E)

    unit = NW * GC
    nchunks = min(5, E // unit)
    tu = E // unit
    cuts = [round(i * tu / nchunks) * unit for i in range(nchunks + 1)]
    bounds = list(zip(cuts[:-1], cuts[1:]))
    chunks = []
    for lo, hi in bounds:
        eh = hi - lo
        epw = eh // NW
        chunks.append(dict(
            epw=epw,
            e3=eidx[lo:hi].reshape(NW, epw // GC, GC),
            d3=dst[lo:hi].reshape(NW, epw // GC, GC),
            rel3=rel_ids[lo:hi].reshape(eh // EB, 1, EB),
        ))

    h = node_states
    L = msg_W1.shape[0]
    for l in range(L):
        wall = jnp.concatenate(
            [msg_W1[l][:D], gate_W1[l][D:2 * D], gate_W1[l][:D]], axis=1)
        wrel = jnp.concatenate([msg_W1[l][D:], gate_W1[l][2 * D:]], axis=1)
        brel = jnp.concatenate([msg_b1[l], gate_b1[l]])[None, :]
        t, pgd, reltab = _proj_call(h, wall, rel8, wrel, brel,
                                    msg_W2[l].astype(jnp.bfloat16),
                                    msg_b2[l][None, :])
        parts = []
        for ck in chunks:
            gsrc, gdst = _gather_call(t, pgd, ck["e3"], ck["d3"], ck["epw"])
            ge = _edge_call(gsrc, gdst, ck["rel3"], reltab,
                            gate_W2[l].T, gate_b2[l][None, :])
            parts.append(_scatter_call(ge, ck["d3"], zeros_nd, ck["epw"]))
        h = _ln_call(h, parts, ln_g[l][None, :], ln_b[l][None, :])

    pooled = _pool_call(h, pool_W1, pool_b1[None, :], pool_W2, pool_b2[None, :])
    return jnp.concatenate([h, pooled[:1]], axis=0)
